# Initial kernel scaffold; baseline (speedup 1.0000x reference)
import jax
import jax.numpy as jnp
from jax.experimental import pallas as pl


def kernel(x, edge_index, etypes, history_map, history_buffer, history_size, W, loop_w, b):
    raise NotImplementedError("write your pallas kernel here")


# speedup vs baseline: 9.5136x; 9.5136x over previous
import functools

import jax
import jax.numpy as jnp
from jax import lax
from jax.experimental import pallas as pl
from jax.experimental.pallas import tpu as pltpu
from jax.experimental.pallas import tpu_sc as plsc


def _mm_body(x_ref, w_ref, b_ref, o_ref):
    o_ref[0] = (
        jnp.dot(x_ref[...], w_ref[0], preferred_element_type=jnp.float32)
        + b_ref[0, 0][None, :]
    )


def _sum_body(p_ref, sl_ref, o_ref):
    o_ref[...] = p_ref[0] + p_ref[1] + sl_ref[0]


def kernel(x, edge_index, etypes, history_map, history_buffer, history_size, W, loop_w, b):
    N, D_IN = x.shape
    R = W.shape[0]
    E = etypes.shape[0]
    H, D_OUT = history_buffer.shape

    NC, NS = 2, 16
    NW = NC * NS
    CH = 128
    TN = 400

    src = edge_index[0]
    dst = edge_index[1]

    Wfull = jnp.concatenate([W, loop_w[None]], axis=0)
    bias_full = jnp.zeros((R + 1, 1, D_OUT), jnp.float32).at[R, 0].set(b)
    xw = pl.pallas_call(
        _mm_body,
        grid=(R + 1, N // TN),
        in_specs=[
            pl.BlockSpec((TN, D_IN), lambda r, i: (i, 0)),
            pl.BlockSpec((1, D_IN, D_OUT), lambda r, i: (r, 0, 0)),
            pl.BlockSpec((1, 1, D_OUT), lambda r, i: (r, 0, 0)),
        ],
        out_specs=pl.BlockSpec((1, TN, D_OUT), lambda r, i: (r, i, 0)),
        out_shape=jax.ShapeDtypeStruct((R + 1, N, D_OUT), jnp.float32),
    )(x, Wfull, bias_full)
    xw_flat = xw.reshape(((R + 1) * N, D_OUT))

    per_tile = -(-E // (NW * CH)) * CH
    Epad = per_tile * NW
    pad = Epad - E
    gidx = etypes * N + src
    gidx_p = jnp.concatenate([gidx, jnp.zeros((pad,), jnp.int32)])
    dst_p = jnp.concatenate([dst, jnp.full((pad,), N, jnp.int32)])
    NACC = N + 8
    ZCH = 640
    ZST = 624
    zeros_blk = jnp.zeros((ZCH, D_OUT), jnp.float32)

    mesh = plsc.VectorSubcoreMesh(core_axis_name="c", subcore_axis_name="s")

    @functools.partial(
        pl.kernel,
        out_type=jax.ShapeDtypeStruct((NC, N, D_OUT), jnp.float32),
        mesh=mesh,
        scratch_types=[
            pltpu.VMEM((CH,), jnp.int32),
            pltpu.VMEM((CH,), jnp.int32),
            pltpu.VMEM((CH, D_OUT), jnp.float32),
            pltpu.VMEM_SHARED((NACC, D_OUT), jnp.float32),
            pltpu.SemaphoreType.DMA,
        ],
    )
    def edge_agg(xw_hbm, gidx_hbm, dst_hbm, z_hbm, out_hbm, gi_v, di_v, rows_v, acc_sh, sem):
        c = lax.axis_index("c")
        s = lax.axis_index("s")
        wid = s * NC + c
        zbase = ZST * s
        pltpu.sync_copy(z_hbm, acc_sh.at[pl.ds(zbase, ZCH)])
        plsc.subcore_barrier()
        ebase = wid * per_tile

        def body(i, carry):
            off = ebase + i * CH
            pltpu.sync_copy(gidx_hbm.at[pl.ds(off, CH)], gi_v)
            pltpu.sync_copy(dst_hbm.at[pl.ds(off, CH)], di_v)
            pltpu.async_copy(xw_hbm.at[gi_v], rows_v, sem).wait()
            pltpu.sync_copy(rows_v, acc_sh.at[di_v], add=True)
            return carry

        lax.fori_loop(0, per_tile // CH, body, 0)
        plsc.subcore_barrier()
        pltpu.sync_copy(acc_sh.at[pl.ds(zbase, ZCH)], out_hbm.at[c, pl.ds(zbase, ZCH)])

    partials = edge_agg(xw_flat, gidx_p, dst_p, zeros_blk)

    outpre = pl.pallas_call(
        _sum_body,
        grid=(N // TN,),
        in_specs=[
            pl.BlockSpec((NC, TN, D_OUT), lambda i: (0, i, 0)),
            pl.BlockSpec((1, TN, D_OUT), lambda i: (R, i, 0)),
        ],
        out_specs=pl.BlockSpec((TN, D_OUT), lambda i: (i, 0)),
        out_shape=jax.ShapeDtypeStruct((N, D_OUT), jnp.float32),
    )(partials, xw)

    combined = jnp.concatenate([history_buffer, outpre], axis=0)
    valid = (history_map != -1) & (history_size != 0)
    sel = jnp.where(valid, history_map, H + jnp.arange(N, dtype=jnp.int32))
    BPW = 320
    BP = BPW * NW
    sel_p = jnp.concatenate([sel, jnp.zeros((BP - N,), jnp.int32)])
    G = 80

    @functools.partial(
        pl.kernel,
        out_type=jax.ShapeDtypeStruct((BP, D_OUT), jnp.float32),
        mesh=mesh,
        scratch_types=[
            pltpu.VMEM((BPW,), jnp.int32),
            pltpu.VMEM((G, D_OUT), jnp.float32),
            pltpu.SemaphoreType.DMA,
        ],
    )
    def hist_gather(comb_hbm, sel_hbm, out_hbm, idx_v, rows_v, sem):
        c = lax.axis_index("c")
        s = lax.axis_index("s")
        wid = s * NC + c
        base = wid * BPW
        pltpu.sync_copy(sel_hbm.at[pl.ds(base, BPW)], idx_v)
        for k in range(BPW // G):
            pltpu.async_copy(comb_hbm.at[idx_v.at[pl.ds(k * G, G)]], rows_v, sem).wait()
            pltpu.sync_copy(rows_v, out_hbm.at[pl.ds(base + k * G, G)])

    outp = hist_gather(combined, sel_p)
    out = outp[:N]
    return (out, out)


# speedup vs baseline: 17.1036x; 1.7978x over previous
import functools

import jax
import jax.numpy as jnp
from jax import lax
from jax.experimental import pallas as pl
from jax.experimental.pallas import tpu as pltpu
from jax.experimental.pallas import tpu_sc as plsc


def _mm_body(x_ref, w_ref, b_ref, o_ref):
    o_ref[...] = (
        jnp.dot(x_ref[...], w_ref[...], preferred_element_type=jnp.float32)
        + b_ref[0][None, :]
    )


def _sum_body(p_ref, sl_ref, o_ref):
    o_ref[...] = p_ref[0] + sl_ref[...]


def kernel(x, edge_index, etypes, history_map, history_buffer, history_size, W, loop_w, b):
    N, D_IN = x.shape
    R = W.shape[0]
    E = etypes.shape[0]
    H, D_OUT = history_buffer.shape
    RP = R + 1

    NC, NS = 2, 16
    NW = NC * NS
    CH = 128
    NBUF = 2
    TM = 2000
    NLOC = N // NC
    NACC = NLOC + 8
    TRASH = NLOC

    src = edge_index[0]
    dst = edge_index[1]

    Wflat = jnp.transpose(
        jnp.concatenate([W, loop_w[None]], axis=0), (1, 0, 2)
    ).reshape(D_IN, RP * D_OUT)
    bias_row = jnp.concatenate([jnp.zeros((R * D_OUT,), jnp.float32), b])[None, :]
    xw = pl.pallas_call(
        _mm_body,
        grid=(N // TM,),
        in_specs=[
            pl.BlockSpec((TM, D_IN), lambda i: (i, 0)),
            pl.BlockSpec((D_IN, RP * D_OUT), lambda i: (0, 0)),
            pl.BlockSpec((1, RP * D_OUT), lambda i: (0, 0)),
        ],
        out_specs=pl.BlockSpec((TM, RP * D_OUT), lambda i: (i, 0)),
        out_shape=jax.ShapeDtypeStruct((N, RP * D_OUT), jnp.float32),
    )(x, Wflat, bias_row)
    xw_flat = xw.reshape((N * RP, D_OUT))

    nch = -(-(-(-E // (NS * CH))) // NBUF) * NBUF
    ncw = nch * CH
    Epad = ncw * NS
    pad = Epad - E
    gidx = src * RP + etypes
    gidx_p = jnp.concatenate([gidx, jnp.zeros((pad,), jnp.int32)]).reshape(NS, ncw)
    dst_p = jnp.concatenate([dst, jnp.full((pad,), -1, jnp.int32)]).reshape(NS, ncw)
    VT = 5120
    valid_i = ((history_map != -1) & (history_size != 0)).astype(jnp.int32)
    vpad = jnp.concatenate([valid_i, jnp.zeros((NC * VT - N,), jnp.int32)])
    vt = jnp.stack([
        lax.dynamic_slice(vpad, (c * NLOC,), (VT,)) for c in range(NC)
    ]).reshape(NC, VT // 128, 128)
    ZCH = 320
    ZST = 312
    zeros_blk = jnp.zeros((ZCH, D_OUT), jnp.float32)
    G16 = ncw // 16

    mesh = plsc.VectorSubcoreMesh(
        core_axis_name="c", subcore_axis_name="s", num_cores=NC, num_subcores=NS
    )

    @functools.partial(
        pl.kernel,
        out_type=jax.ShapeDtypeStruct((NC, NLOC, D_OUT), jnp.float32),
        mesh=mesh,
        compiler_params=pltpu.CompilerParams(needs_layout_passes=False),
        scratch_types=[
            pltpu.VMEM((ncw + CH,), jnp.int32),
            pltpu.VMEM((ncw + CH,), jnp.int32),
            pltpu.VMEM((VT // 128, 128), jnp.int32),
            pltpu.VMEM((NBUF, CH), jnp.int32),
            pltpu.VMEM((NBUF, CH, D_OUT), jnp.float32),
            pltpu.VMEM_SHARED((NACC, D_OUT), jnp.float32),
            pltpu.SemaphoreType.DMA,
            pltpu.SemaphoreType.DMA,
            pltpu.SemaphoreType.DMA,
        ]
        + [pltpu.SemaphoreType.DMA] * (2 * NBUF),
    )
    def edge_agg(xw_hbm, gidx_hbm, dst_hbm, vt_hbm, z_hbm, out_hbm,
                 gi_v, di_v, vld_v, dstg_v, rows_v, acc_sh, si0, si1, si2, *sems):
        sg = sems[:NBUF]
        ss = sems[NBUF:]
        c = lax.axis_index("c")
        s = lax.axis_index("s")
        lo = c * NLOC
        cp0 = pltpu.async_copy(gidx_hbm.at[s], gi_v.at[pl.ds(0, ncw)], si0)
        cp1 = pltpu.async_copy(dst_hbm.at[s], di_v.at[pl.ds(0, ncw)], si1)
        cp2 = pltpu.async_copy(vt_hbm.at[c], vld_v, si2)
        zbase = ZST * s
        pltpu.sync_copy(z_hbm, acc_sh.at[pl.ds(zbase, ZCH)])
        cp0.wait()
        cp1.wait()
        cp2.wait()

        def comp(g, off):
            d = di_v[pl.ds(g * 16, 16)]
            gx = gi_v[pl.ds(g * 16, 16)]
            dl = d - lo
            inr = (dl >= 0) & (dl < NLOC)
            dls = jnp.where(inr, dl, 0)
            hv = plsc.load_gather(vld_v, [dls >> 7, dls & 127])
            keep = inr & (hv == 0)
            plsc.store_compressed(di_v.at[pl.ds(off, 16)], dl, mask=keep)
            plsc.store_compressed(gi_v.at[pl.ds(off, 16)], gx, mask=keep)
            return off + jnp.sum(keep.astype(jnp.int32))

        off = lax.fori_loop(0, G16, comp, jnp.int32(0))
        for k in range(CH // 16):
            gi_v[pl.ds(off + k * 16, 16)] = jnp.zeros((16,), jnp.int32)
            di_v[pl.ds(off + k * 16, 16)] = jnp.full((16,), TRASH, jnp.int32)
        ncc = (off + CH - 1) // CH
        nb = (ncc + NBUF - 1) // NBUF
        plsc.subcore_barrier()

        def fire_gather(j, bslot):
            pltpu.async_copy(xw_hbm.at[gi_v.at[pl.ds(j * CH, CH)]],
                             rows_v.at[bslot], sg[bslot])

        def fire_scatter(j, bslot):
            for k in range(CH // 16):
                dstg_v[bslot, pl.ds(k * 16, 16)] = di_v[pl.ds(j * CH + k * 16, 16)]
            pltpu.async_copy(rows_v.at[bslot], acc_sh.at[dstg_v.at[bslot]],
                             ss[bslot], add=True)

        def wait_g(bslot):
            pltpu.make_async_copy(xw_hbm.at[pl.ds(0, CH)], rows_v.at[bslot],
                                  sg[bslot]).wait()

        def wait_s(bslot):
            pltpu.make_async_copy(xw_hbm.at[pl.ds(0, CH)], rows_v.at[bslot],
                                  ss[bslot]).wait()

        def body(j, carry):
            fire_gather(j, 0)
            wait_g(0)
            fire_scatter(j, 0)
            wait_s(0)
            return carry

        lax.fori_loop(0, ncc, body, jnp.int32(0))
        plsc.subcore_barrier()
        pltpu.sync_copy(acc_sh.at[pl.ds(zbase, ZCH)],
                        out_hbm.at[c, pl.ds(zbase, ZCH)])

    partials = edge_agg(xw_flat, gidx_p, dst_p, vt, zeros_blk)

    TNC = 1000
    outpre = pl.pallas_call(
        _sum_body,
        grid=(NC, NLOC // TNC),
        in_specs=[
            pl.BlockSpec((1, TNC, D_OUT), lambda c, i: (c, i, 0)),
            pl.BlockSpec((TNC, D_OUT), lambda c, i: (c * (NLOC // TNC) + i, R)),
        ],
        out_specs=pl.BlockSpec((TNC, D_OUT), lambda c, i: (c * (NLOC // TNC) + i, 0)),
        out_shape=jax.ShapeDtypeStruct((N, D_OUT), jnp.float32),
    )(partials, xw)

    combined = jnp.concatenate([history_buffer, outpre], axis=0)
    valid = (history_map != -1) & (history_size != 0)
    sel = jnp.where(valid, history_map, H + jnp.arange(N, dtype=jnp.int32))
    BPW = 320
    BP = BPW * NW
    sel_p = jnp.concatenate([sel, jnp.zeros((BP - N,), jnp.int32)])
    G = 80

    @functools.partial(
        pl.kernel,
        out_type=jax.ShapeDtypeStruct((BP, D_OUT), jnp.float32),
        mesh=mesh,
        scratch_types=[
            pltpu.VMEM((BPW,), jnp.int32),
            pltpu.VMEM((G, D_OUT), jnp.float32),
            pltpu.SemaphoreType.DMA,
        ],
    )
    def hist_gather(comb_hbm, sel_hbm, out_hbm, idx_v, rows_v, sem):
        c = lax.axis_index("c")
        s = lax.axis_index("s")
        wid = s * NC + c
        base = wid * BPW
        pltpu.sync_copy(sel_hbm.at[pl.ds(base, BPW)], idx_v)
        for k in range(BPW // G):
            pltpu.async_copy(comb_hbm.at[idx_v.at[pl.ds(k * G, G)]], rows_v, sem).wait()
            pltpu.sync_copy(rows_v, out_hbm.at[pl.ds(base + k * G, G)])

    outp = hist_gather(combined, sel_p)
    out = outp[:N]
    return (out, out)


# speedup vs baseline: 17.8911x; 1.0460x over previous
import functools

import jax
import jax.numpy as jnp
from jax import lax
from jax.experimental import pallas as pl
from jax.experimental.pallas import tpu as pltpu
from jax.experimental.pallas import tpu_sc as plsc


def _mm_body(x_ref, w_ref, b_ref, o_ref):
    o_ref[...] = (
        jnp.dot(x_ref[...], w_ref[...], preferred_element_type=jnp.float32)
        + b_ref[0][None, :]
    )


def _sum_body(p_ref, sl_ref, o_ref):
    o_ref[...] = p_ref[0] + sl_ref[...]


def kernel(x, edge_index, etypes, history_map, history_buffer, history_size, W, loop_w, b):
    N, D_IN = x.shape
    R = W.shape[0]
    E = etypes.shape[0]
    H, D_OUT = history_buffer.shape
    RP = R + 1

    NC, NS = 2, 16
    NW = NC * NS
    CH = 128
    NBUF = 2
    TM = 2000
    NLOC = N // NC
    NACC = NLOC + 8
    TRASH = NLOC

    src = edge_index[0]
    dst = edge_index[1]

    Wflat = jnp.transpose(
        jnp.concatenate([W, loop_w[None]], axis=0), (1, 0, 2)
    ).reshape(D_IN, RP * D_OUT)
    bias_row = jnp.concatenate([jnp.zeros((R * D_OUT,), jnp.float32), b])[None, :]
    xw = pl.pallas_call(
        _mm_body,
        grid=(N // TM,),
        in_specs=[
            pl.BlockSpec((TM, D_IN), lambda i: (i, 0)),
            pl.BlockSpec((D_IN, RP * D_OUT), lambda i: (0, 0)),
            pl.BlockSpec((1, RP * D_OUT), lambda i: (0, 0)),
        ],
        out_specs=pl.BlockSpec((TM, RP * D_OUT), lambda i: (i, 0)),
        out_shape=jax.ShapeDtypeStruct((N, RP * D_OUT), jnp.float32),
    )(x, Wflat, bias_row)
    xw_flat = xw.reshape((N * RP, D_OUT))

    nch = -(-(-(-E // (NS * CH))) // NBUF) * NBUF
    ncw = nch * CH
    Epad = ncw * NS
    pad = Epad - E
    gidx = src * RP + etypes
    gidx_p = jnp.concatenate([gidx, jnp.zeros((pad,), jnp.int32)]).reshape(NS, ncw)
    dst_p = jnp.concatenate([dst, jnp.full((pad,), -1, jnp.int32)]).reshape(NS, ncw)
    VT = 5120
    valid_i = ((history_map != -1) & (history_size != 0)).astype(jnp.int32)
    vpad = jnp.concatenate([valid_i, jnp.zeros((NC * VT - N,), jnp.int32)])
    vt = jnp.stack([
        lax.dynamic_slice(vpad, (c * NLOC,), (VT,)) for c in range(NC)
    ]).reshape(NC, VT // 128, 128)
    ZCH = 320
    ZST = 312
    zeros_blk = jnp.zeros((ZCH, D_OUT), jnp.float32)
    G16 = ncw // 16

    mesh = plsc.VectorSubcoreMesh(
        core_axis_name="c", subcore_axis_name="s", num_cores=NC, num_subcores=NS
    )

    @functools.partial(
        pl.kernel,
        out_type=jax.ShapeDtypeStruct((NC, NLOC, D_OUT), jnp.float32),
        mesh=mesh,
        compiler_params=pltpu.CompilerParams(needs_layout_passes=False),
        scratch_types=[
            pltpu.VMEM((ncw + CH,), jnp.int32),
            pltpu.VMEM((ncw + CH,), jnp.int32),
            pltpu.VMEM((VT // 128, 128), jnp.int32),
            pltpu.VMEM((NBUF, CH), jnp.int32),
            pltpu.VMEM((NBUF, CH, D_OUT), jnp.float32),
            pltpu.VMEM_SHARED((NACC, D_OUT), jnp.float32),
            pltpu.SemaphoreType.DMA,
            pltpu.SemaphoreType.DMA,
            pltpu.SemaphoreType.DMA,
        ]
        + [pltpu.SemaphoreType.DMA] * (2 * NBUF),
    )
    def edge_agg(xw_hbm, gidx_hbm, dst_hbm, vt_hbm, z_hbm, out_hbm,
                 gi_v, di_v, vld_v, dstg_v, rows_v, acc_sh, si0, si1, si2, *sems):
        sg = sems[:NBUF]
        ss = sems[NBUF:]
        c = lax.axis_index("c")
        s = lax.axis_index("s")
        lo = c * NLOC
        cp0 = pltpu.async_copy(gidx_hbm.at[s], gi_v.at[pl.ds(0, ncw)], si0)
        cp1 = pltpu.async_copy(dst_hbm.at[s], di_v.at[pl.ds(0, ncw)], si1)
        cp2 = pltpu.async_copy(vt_hbm.at[c], vld_v, si2)
        zbase = ZST * s
        pltpu.sync_copy(z_hbm, acc_sh.at[pl.ds(zbase, ZCH)])
        cp0.wait()
        cp1.wait()
        cp2.wait()

        def comp(g, off):
            d = di_v[pl.ds(g * 16, 16)]
            gx = gi_v[pl.ds(g * 16, 16)]
            dl = d - lo
            inr = (dl >= 0) & (dl < NLOC)
            dls = jnp.where(inr, dl, 0)
            hv = plsc.load_gather(vld_v, [dls >> 7, dls & 127])
            keep = inr & (hv == 0)
            plsc.store_compressed(di_v.at[pl.ds(off, 16)], dl, mask=keep)
            plsc.store_compressed(gi_v.at[pl.ds(off, 16)], gx, mask=keep)
            return off + jnp.sum(keep.astype(jnp.int32))

        off = lax.fori_loop(0, G16, comp, jnp.int32(0))
        for k in range(CH // 16):
            gi_v[pl.ds(off + k * 16, 16)] = jnp.zeros((16,), jnp.int32)
            di_v[pl.ds(off + k * 16, 16)] = jnp.full((16,), TRASH, jnp.int32)
        ncc = (off + CH - 1) // CH
        nb = (ncc + NBUF - 1) // NBUF
        plsc.subcore_barrier()

        def fire_gather(j, bslot):
            pltpu.async_copy(xw_hbm.at[gi_v.at[pl.ds(j * CH, CH)]],
                             rows_v.at[bslot], sg[bslot])

        def fire_scatter(j, bslot):
            for k in range(CH // 16):
                dstg_v[bslot, pl.ds(k * 16, 16)] = di_v[pl.ds(j * CH + k * 16, 16)]
            pltpu.async_copy(rows_v.at[bslot], acc_sh.at[dstg_v.at[bslot]],
                             ss[bslot], add=True)

        def wait_g(bslot):
            pltpu.make_async_copy(xw_hbm.at[pl.ds(0, CH)], rows_v.at[bslot],
                                  sg[bslot]).wait()

        def wait_s(bslot):
            pltpu.make_async_copy(xw_hbm.at[pl.ds(0, CH)], rows_v.at[bslot],
                                  ss[bslot]).wait()

        for bslot in range(NBUF):
            @pl.when(bslot < ncc)
            def _(bslot=bslot):
                fire_gather(bslot, bslot)

        def body(blk, carry):
            for bslot in range(NBUF):
                j = (blk - 1) * NBUF + bslot

                @pl.when(j < ncc)
                def _(j=j, bslot=bslot):
                    wait_g(bslot)
                    fire_scatter(j, bslot)
            for bslot in range(NBUF):
                j = blk * NBUF + bslot

                @pl.when(j < ncc)
                def _(j=j, bslot=bslot):
                    wait_s(bslot)
                    fire_gather(j, bslot)
            return carry

        nb = (ncc + NBUF - 1) // NBUF
        lax.fori_loop(1, nb, body, jnp.int32(0))
        base = (nb - 1) * NBUF
        for bslot in range(NBUF):
            j = base + bslot

            @pl.when((j >= 0) & (j < ncc))
            def _(j=j, bslot=bslot):
                wait_g(bslot)
                fire_scatter(j, bslot)
        for bslot in range(NBUF):
            j = base + bslot

            @pl.when((j >= 0) & (j < ncc))
            def _(j=j, bslot=bslot):
                wait_s(bslot)
        plsc.subcore_barrier()
        pltpu.sync_copy(acc_sh.at[pl.ds(zbase, ZCH)],
                        out_hbm.at[c, pl.ds(zbase, ZCH)])

    partials = edge_agg(xw_flat, gidx_p, dst_p, vt, zeros_blk)

    TNC = 1000
    outpre = pl.pallas_call(
        _sum_body,
        grid=(NC, NLOC // TNC),
        in_specs=[
            pl.BlockSpec((1, TNC, D_OUT), lambda c, i: (c, i, 0)),
            pl.BlockSpec((TNC, D_OUT), lambda c, i: (c * (NLOC // TNC) + i, R)),
        ],
        out_specs=pl.BlockSpec((TNC, D_OUT), lambda c, i: (c * (NLOC // TNC) + i, 0)),
        out_shape=jax.ShapeDtypeStruct((N, D_OUT), jnp.float32),
    )(partials, xw)

    combined = jnp.concatenate([history_buffer, outpre], axis=0)
    valid = (history_map != -1) & (history_size != 0)
    sel = jnp.where(valid, history_map, H + jnp.arange(N, dtype=jnp.int32))
    BPW = 320
    BP = BPW * NW
    sel_p = jnp.concatenate([sel, jnp.zeros((BP - N,), jnp.int32)])
    G = 80

    @functools.partial(
        pl.kernel,
        out_type=jax.ShapeDtypeStruct((BP, D_OUT), jnp.float32),
        mesh=mesh,
        scratch_types=[
            pltpu.VMEM((BPW,), jnp.int32),
            pltpu.VMEM((G, D_OUT), jnp.float32),
            pltpu.SemaphoreType.DMA,
        ],
    )
    def hist_gather(comb_hbm, sel_hbm, out_hbm, idx_v, rows_v, sem):
        c = lax.axis_index("c")
        s = lax.axis_index("s")
        wid = s * NC + c
        base = wid * BPW
        pltpu.sync_copy(sel_hbm.at[pl.ds(base, BPW)], idx_v)
        for k in range(BPW // G):
            pltpu.async_copy(comb_hbm.at[idx_v.at[pl.ds(k * G, G)]], rows_v, sem).wait()
            pltpu.sync_copy(rows_v, out_hbm.at[pl.ds(base + k * G, G)])

    outp = hist_gather(combined, sel_p)
    out = outp[:N]
    return (out, out)


# speedup vs baseline: 19.3581x; 1.0820x over previous
import functools

import jax
import jax.numpy as jnp
from jax import lax
from jax.experimental import pallas as pl
from jax.experimental.pallas import tpu as pltpu
from jax.experimental.pallas import tpu_sc as plsc


def _mm_body(x_ref, w_ref, b_ref, o_ref):
    o_ref[0] = (
        jnp.dot(x_ref[...], w_ref[0], preferred_element_type=jnp.float32)
        + b_ref[0, 0][None, :]
    )


def kernel(x, edge_index, etypes, history_map, history_buffer, history_size, W, loop_w, b):
    N, D_IN = x.shape
    R = W.shape[0]
    E = etypes.shape[0]
    H, D_OUT = history_buffer.shape
    RP = R + 1

    NC, NS = 2, 16
    CH = 128
    NBUF = 2
    TM = 2000
    NLOC = N // NC
    NACC = NLOC + 8
    TRASH = NLOC
    MT = 5120

    src = edge_index[0]
    dst = edge_index[1]

    Wfull = jnp.concatenate([W, loop_w[None]], axis=0)
    bias3 = jnp.zeros((RP, 1, D_OUT), jnp.float32).at[R, 0].set(b)
    xw = pl.pallas_call(
        _mm_body,
        grid=(RP, N // TM),
        in_specs=[
            pl.BlockSpec((TM, D_IN), lambda r, i: (i, 0)),
            pl.BlockSpec((1, D_IN, D_OUT), lambda r, i: (r, 0, 0)),
            pl.BlockSpec((1, 1, D_OUT), lambda r, i: (r, 0, 0)),
        ],
        out_specs=pl.BlockSpec((1, TM, D_OUT), lambda r, i: (r, i, 0)),
        out_shape=jax.ShapeDtypeStruct((RP, N, D_OUT), jnp.float32),
    )(x, Wfull, bias3)
    xw_flat = xw.reshape((RP * N, D_OUT))

    nch = -(-(-(-E // (NS * CH))) // NBUF) * NBUF
    ncw = nch * CH
    Epad = ncw * NS
    pad = Epad - E
    gidx = etypes * N + src
    gidx_p = jnp.concatenate([gidx, jnp.zeros((pad,), jnp.int32)]).reshape(NS, ncw)
    dst_p = jnp.concatenate([dst, jnp.full((pad,), -1, jnp.int32)]).reshape(NS, ncw)
    hm_eff = jnp.where(history_size != 0, history_map, -1)
    mp_pad = jnp.concatenate([hm_eff, jnp.full((NC * MT - N,), -1, jnp.int32)])
    mp2 = jnp.stack([
        lax.dynamic_slice(mp_pad, (c * NLOC,), (MT,)) for c in range(NC)
    ])
    ZCH = 320
    ZST = 312
    zeros_blk = jnp.zeros((ZCH, D_OUT), jnp.float32)
    G16 = ncw // 16
    HB = 448

    mesh = plsc.VectorSubcoreMesh(
        core_axis_name="c", subcore_axis_name="s", num_cores=NC, num_subcores=NS
    )

    @functools.partial(
        pl.kernel,
        out_type=jax.ShapeDtypeStruct((N, D_OUT), jnp.float32),
        mesh=mesh,
        compiler_params=pltpu.CompilerParams(needs_layout_passes=False),
        scratch_types=[
            pltpu.VMEM((ncw + CH,), jnp.int32),
            pltpu.VMEM((ncw + CH,), jnp.int32),
            pltpu.VMEM((MT,), jnp.int32),
            pltpu.VMEM((NBUF, CH), jnp.int32),
            pltpu.VMEM((NBUF, CH, D_OUT), jnp.float32),
            pltpu.VMEM((HB,), jnp.int32),
            pltpu.VMEM((HB,), jnp.int32),
            pltpu.VMEM_SHARED((NACC, D_OUT), jnp.float32),
            pltpu.SemaphoreType.DMA,
            pltpu.SemaphoreType.DMA,
            pltpu.SemaphoreType.DMA,
        ]
        + [pltpu.SemaphoreType.DMA] * (2 * NBUF),
    )
    def edge_agg(xw_hbm, gidx_hbm, dst_hbm, mp_hbm, z_hbm, hist_hbm, out_hbm,
                 gi_v, di_v, map_v, dstg_v, rows_v, hi_v, ho_v, acc_sh,
                 si0, si1, si2, *sems):
        sg = sems[:NBUF]
        ss = sems[NBUF:]
        c = lax.axis_index("c")
        s = lax.axis_index("s")
        lo = c * NLOC
        cp0 = pltpu.async_copy(gidx_hbm.at[s], gi_v.at[pl.ds(0, ncw)], si0)
        cp1 = pltpu.async_copy(dst_hbm.at[s], di_v.at[pl.ds(0, ncw)], si1)
        cp2 = pltpu.async_copy(mp_hbm.at[c], map_v, si2)
        zbase = ZST * s
        pltpu.sync_copy(z_hbm, acc_sh.at[pl.ds(zbase, ZCH)])
        cp0.wait()
        cp1.wait()
        cp2.wait()

        def comp(g, off):
            d = di_v[pl.ds(g * 16, 16)]
            gx = gi_v[pl.ds(g * 16, 16)]
            dl = d - lo
            inr = (dl >= 0) & (dl < NLOC)
            dls = jnp.where(inr, dl, 0)
            hv = plsc.load_gather(map_v, [dls])
            keep = inr & (hv == -1)
            plsc.store_compressed(di_v.at[pl.ds(off, 16)], dl, mask=keep)
            plsc.store_compressed(gi_v.at[pl.ds(off, 16)], gx, mask=keep)
            return off + jnp.sum(keep.astype(jnp.int32))

        off = lax.fori_loop(0, G16, comp, jnp.int32(0))
        for k in range(CH // 16):
            gi_v[pl.ds(off + k * 16, 16)] = jnp.zeros((16,), jnp.int32)
            di_v[pl.ds(off + k * 16, 16)] = jnp.full((16,), TRASH, jnp.int32)
        ncc = (off + CH - 1) // CH
        plsc.subcore_barrier()

        def fire_gather(j, bslot):
            pltpu.async_copy(xw_hbm.at[gi_v.at[pl.ds(j * CH, CH)]],
                             rows_v.at[bslot], sg[bslot])

        def fire_scatter(j, bslot):
            for k in range(CH // 16):
                dstg_v[bslot, pl.ds(k * 16, 16)] = di_v[pl.ds(j * CH + k * 16, 16)]
            pltpu.async_copy(rows_v.at[bslot], acc_sh.at[dstg_v.at[bslot]],
                             ss[bslot], add=True)

        def wait_g(bslot):
            pltpu.make_async_copy(xw_hbm.at[pl.ds(0, CH)], rows_v.at[bslot],
                                  sg[bslot]).wait()

        def wait_s(bslot):
            pltpu.make_async_copy(xw_hbm.at[pl.ds(0, CH)], rows_v.at[bslot],
                                  ss[bslot]).wait()

        for bslot in range(NBUF):
            @pl.when(bslot < ncc)
            def _(bslot=bslot):
                fire_gather(bslot, bslot)

        def body(blk, carry):
            for bslot in range(NBUF):
                j = (blk - 1) * NBUF + bslot

                @pl.when(j < ncc)
                def _(j=j, bslot=bslot):
                    wait_g(bslot)
                    fire_scatter(j, bslot)
            for bslot in range(NBUF):
                j = blk * NBUF + bslot

                @pl.when(j < ncc)
                def _(j=j, bslot=bslot):
                    wait_s(bslot)
                    fire_gather(j, bslot)
            return carry

        nb = (ncc + NBUF - 1) // NBUF
        lax.fori_loop(1, nb, body, jnp.int32(0))
        base = (nb - 1) * NBUF
        for bslot in range(NBUF):
            j = base + bslot

            @pl.when((j >= 0) & (j < ncc))
            def _(j=j, bslot=bslot):
                wait_g(bslot)
                fire_scatter(j, bslot)
        for bslot in range(NBUF):
            j = base + bslot

            @pl.when((j >= 0) & (j < ncc))
            def _(j=j, bslot=bslot):
                wait_s(bslot)
        plsc.subcore_barrier()

        g0 = lo + zbase
        for q in range(ZCH // 80):
            pltpu.sync_copy(acc_sh.at[pl.ds(zbase + 80 * q, 80)],
                            rows_v.at[0, pl.ds(0, 80)])
            pltpu.sync_copy(xw_hbm.at[pl.ds(R * N + g0 + 80 * q, 80)],
                            rows_v.at[1, pl.ds(0, 80)])

            def addrow(i, carry):
                for l in range(D_OUT // 16):
                    rows_v[0, i, pl.ds(l * 16, 16)] = (
                        rows_v[0, i, pl.ds(l * 16, 16)]
                        + rows_v[1, i, pl.ds(l * 16, 16)]
                    )
                return carry

            lax.fori_loop(0, 80, addrow, jnp.int32(0))
            pltpu.sync_copy(rows_v.at[0, pl.ds(0, 80)],
                            out_hbm.at[pl.ds(g0 + 80 * q, 80)])
        plsc.subcore_barrier()

        def hcomp(g, cnt):
            m = map_v[pl.ds(zbase + g * 16, 16)]
            rowv = g0 + g * 16 + lax.iota(jnp.int32, 16)
            vm = m != -1
            plsc.store_compressed(hi_v.at[pl.ds(cnt, 16)], m, mask=vm)
            plsc.store_compressed(ho_v.at[pl.ds(cnt, 16)], rowv, mask=vm)
            return cnt + jnp.sum(vm.astype(jnp.int32))

        cnt = lax.fori_loop(0, ZCH // 16, hcomp, jnp.int32(0))

        @pl.when(cnt > 0)
        def _():
            lasth = hi_v[pl.ds(cnt - 1, 16)][0]
            lasto = ho_v[pl.ds(cnt - 1, 16)][0]
            for k in range(CH // 16):
                hi_v[pl.ds(cnt + k * 16, 16)] = jnp.full((16,), 0, jnp.int32) + lasth
                ho_v[pl.ds(cnt + k * 16, 16)] = jnp.full((16,), 0, jnp.int32) + lasto

            def hbody(j, carry):
                cph = pltpu.async_copy(
                    hist_hbm.at[hi_v.at[pl.ds(j * CH, CH)]], rows_v.at[0], sg[0])
                cph.wait()
                for k in range(CH // 16):
                    dstg_v[0, pl.ds(k * 16, 16)] = ho_v[pl.ds(j * CH + k * 16, 16)]
                pltpu.async_copy(rows_v.at[0], out_hbm.at[dstg_v.at[0]],
                                 ss[0]).wait()
                return carry

            nhc = (cnt + CH - 1) // CH
            lax.fori_loop(0, nhc, hbody, jnp.int32(0))

    out = edge_agg(xw_flat, gidx_p, dst_p, mp2, zeros_blk, history_buffer)
    return (out, out)


# speedup vs baseline: 20.2093x; 1.0440x over previous
import functools

import jax
import jax.numpy as jnp
from jax import lax
from jax.experimental import pallas as pl
from jax.experimental.pallas import tpu as pltpu
from jax.experimental.pallas import tpu_sc as plsc


def _mm_body(x_ref, w_ref, b_ref, o_ref):
    o_ref[0] = (
        jnp.dot(x_ref[...], w_ref[0], preferred_element_type=jnp.float32)
        + b_ref[0, 0][None, :]
    )


def kernel(x, edge_index, etypes, history_map, history_buffer, history_size, W, loop_w, b):
    N, D_IN = x.shape
    R = W.shape[0]
    E = etypes.shape[0]
    H, D_OUT = history_buffer.shape
    RP = R + 1

    NC, NS = 2, 16
    CH = 128
    NBUF = 2
    TM = 2000
    NLOC = N // NC
    NACC = NLOC + 8
    TRASH = NLOC
    MT = 5120

    src = edge_index[0]
    dst = edge_index[1]

    Wfull = jnp.concatenate([W, loop_w[None]], axis=0)
    bias3 = jnp.zeros((RP, 1, D_OUT), jnp.float32).at[R, 0].set(b)
    xw = pl.pallas_call(
        _mm_body,
        grid=(N // TM, RP),
        in_specs=[
            pl.BlockSpec((TM, D_IN), lambda i, r: (i, 0)),
            pl.BlockSpec((1, D_IN, D_OUT), lambda i, r: (r, 0, 0)),
            pl.BlockSpec((1, 1, D_OUT), lambda i, r: (r, 0, 0)),
        ],
        out_specs=pl.BlockSpec((1, TM, D_OUT), lambda i, r: (r, i, 0)),
        out_shape=jax.ShapeDtypeStruct((RP, N, D_OUT), jnp.float32),
    )(x, Wfull, bias3)
    xw_flat = xw.reshape((RP * N, D_OUT))

    nch = -(-(-(-E // (NS * CH))) // NBUF) * NBUF
    ncw = nch * CH
    Epad = ncw * NS
    pad = Epad - E
    gidx = etypes * N + src
    gidx_p = jnp.concatenate([gidx, jnp.zeros((pad,), jnp.int32)]).reshape(NS, ncw)
    dst_p = jnp.concatenate([dst, jnp.full((pad,), -1, jnp.int32)]).reshape(NS, ncw)
    hm_eff = jnp.where(history_size != 0, history_map, -1)
    mp_pad = jnp.concatenate(
        [hm_eff, jnp.full((NLOC + MT - N,), -1, jnp.int32)])
    ZCH = 320
    ZST = 312
    zeros_blk = jnp.zeros((ZCH, D_OUT), jnp.float32)
    G16 = ncw // 16
    HB = 448

    mesh = plsc.VectorSubcoreMesh(
        core_axis_name="c", subcore_axis_name="s", num_cores=NC, num_subcores=NS
    )

    @functools.partial(
        pl.kernel,
        out_type=jax.ShapeDtypeStruct((N, D_OUT), jnp.float32),
        mesh=mesh,
        compiler_params=pltpu.CompilerParams(needs_layout_passes=False),
        scratch_types=[
            pltpu.VMEM((ncw + CH,), jnp.int32),
            pltpu.VMEM((ncw + CH,), jnp.int32),
            pltpu.VMEM((MT,), jnp.int32),
            pltpu.VMEM((NBUF, CH), jnp.int32),
            pltpu.VMEM((NBUF, CH, D_OUT), jnp.float32),
            pltpu.VMEM((HB,), jnp.int32),
            pltpu.VMEM((HB,), jnp.int32),
            pltpu.VMEM_SHARED((NACC, D_OUT), jnp.float32),
            pltpu.SemaphoreType.DMA,
            pltpu.SemaphoreType.DMA,
            pltpu.SemaphoreType.DMA,
        ]
        + [pltpu.SemaphoreType.DMA] * (2 * NBUF),
    )
    def edge_agg(xw_hbm, gidx_hbm, dst_hbm, mp_hbm, z_hbm, hist_hbm, out_hbm,
                 gi_v, di_v, map_v, dstg_v, rows_v, hi_v, ho_v, acc_sh,
                 si0, si1, si2, *sems):
        sg = sems[:NBUF]
        ss = sems[NBUF:]
        c = lax.axis_index("c")
        s = lax.axis_index("s")
        lo = c * NLOC
        cp0 = pltpu.async_copy(gidx_hbm.at[s], gi_v.at[pl.ds(0, ncw)], si0)
        cp1 = pltpu.async_copy(dst_hbm.at[s], di_v.at[pl.ds(0, ncw)], si1)
        cp2 = pltpu.async_copy(mp_hbm.at[pl.ds(lo, MT)], map_v, si2)
        zbase = ZST * s
        pltpu.sync_copy(z_hbm, acc_sh.at[pl.ds(zbase, ZCH)])
        cp0.wait()
        cp1.wait()
        cp2.wait()

        def comp(g, off):
            d = di_v[pl.ds(g * 16, 16)]
            gx = gi_v[pl.ds(g * 16, 16)]
            dl = d - lo
            inr = (dl >= 0) & (dl < NLOC)
            dls = jnp.where(inr, dl, 0)
            hv = plsc.load_gather(map_v, [dls])
            keep = inr & (hv == -1)
            plsc.store_compressed(di_v.at[pl.ds(off, 16)], dl, mask=keep)
            plsc.store_compressed(gi_v.at[pl.ds(off, 16)], gx, mask=keep)
            return off + jnp.sum(keep.astype(jnp.int32))

        off = lax.fori_loop(0, G16, comp, jnp.int32(0))
        for k in range(CH // 16):
            gi_v[pl.ds(off + k * 16, 16)] = jnp.zeros((16,), jnp.int32)
            di_v[pl.ds(off + k * 16, 16)] = jnp.full((16,), TRASH, jnp.int32)
        ncc = (off + CH - 1) // CH
        plsc.subcore_barrier()

        def fire_gather(j, bslot):
            pltpu.async_copy(xw_hbm.at[gi_v.at[pl.ds(j * CH, CH)]],
                             rows_v.at[bslot], sg[bslot])

        def fire_scatter(j, bslot):
            for k in range(CH // 16):
                dstg_v[bslot, pl.ds(k * 16, 16)] = di_v[pl.ds(j * CH + k * 16, 16)]
            pltpu.async_copy(rows_v.at[bslot], acc_sh.at[dstg_v.at[bslot]],
                             ss[bslot], add=True)

        def wait_g(bslot):
            pltpu.make_async_copy(xw_hbm.at[pl.ds(0, CH)], rows_v.at[bslot],
                                  sg[bslot]).wait()

        def wait_s(bslot):
            pltpu.make_async_copy(xw_hbm.at[pl.ds(0, CH)], rows_v.at[bslot],
                                  ss[bslot]).wait()

        for bslot in range(NBUF):
            @pl.when(bslot < ncc)
            def _(bslot=bslot):
                fire_gather(bslot, bslot)

        def body(blk, carry):
            for bslot in range(NBUF):
                j = (blk - 1) * NBUF + bslot

                @pl.when(j < ncc)
                def _(j=j, bslot=bslot):
                    wait_g(bslot)
                    fire_scatter(j, bslot)
            for bslot in range(NBUF):
                j = blk * NBUF + bslot

                @pl.when(j < ncc)
                def _(j=j, bslot=bslot):
                    wait_s(bslot)
                    fire_gather(j, bslot)
            return carry

        nb = (ncc + NBUF - 1) // NBUF
        lax.fori_loop(1, nb, body, jnp.int32(0))
        base = (nb - 1) * NBUF
        for bslot in range(NBUF):
            j = base + bslot

            @pl.when((j >= 0) & (j < ncc))
            def _(j=j, bslot=bslot):
                wait_g(bslot)
                fire_scatter(j, bslot)
        for bslot in range(NBUF):
            j = base + bslot

            @pl.when((j >= 0) & (j < ncc))
            def _(j=j, bslot=bslot):
                wait_s(bslot)
        plsc.subcore_barrier()

        g0 = lo + zbase
        for q in range(ZCH // 80):
            pltpu.sync_copy(acc_sh.at[pl.ds(zbase + 80 * q, 80)],
                            rows_v.at[0, pl.ds(0, 80)])
            pltpu.sync_copy(xw_hbm.at[pl.ds(R * N + g0 + 80 * q, 80)],
                            rows_v.at[1, pl.ds(0, 80)])

            def addrow(i, carry):
                for l in range(D_OUT // 16):
                    rows_v[0, i, pl.ds(l * 16, 16)] = (
                        rows_v[0, i, pl.ds(l * 16, 16)]
                        + rows_v[1, i, pl.ds(l * 16, 16)]
                    )
                return carry

            lax.fori_loop(0, 80, addrow, jnp.int32(0))
            pltpu.sync_copy(rows_v.at[0, pl.ds(0, 80)],
                            out_hbm.at[pl.ds(g0 + 80 * q, 80)])
        plsc.subcore_barrier()

        def hcomp(g, cnt):
            m = map_v[pl.ds(zbase + g * 16, 16)]
            rowv = g0 + g * 16 + lax.iota(jnp.int32, 16)
            vm = m != -1
            plsc.store_compressed(hi_v.at[pl.ds(cnt, 16)], m, mask=vm)
            plsc.store_compressed(ho_v.at[pl.ds(cnt, 16)], rowv, mask=vm)
            return cnt + jnp.sum(vm.astype(jnp.int32))

        cnt = lax.fori_loop(0, ZCH // 16, hcomp, jnp.int32(0))

        @pl.when(cnt > 0)
        def _():
            lasth = hi_v[pl.ds(cnt - 1, 16)][0]
            lasto = ho_v[pl.ds(cnt - 1, 16)][0]
            for k in range(CH // 16):
                hi_v[pl.ds(cnt + k * 16, 16)] = jnp.full((16,), 0, jnp.int32) + lasth
                ho_v[pl.ds(cnt + k * 16, 16)] = jnp.full((16,), 0, jnp.int32) + lasto

            def hbody(j, carry):
                cph = pltpu.async_copy(
                    hist_hbm.at[hi_v.at[pl.ds(j * CH, CH)]], rows_v.at[0], sg[0])
                cph.wait()
                for k in range(CH // 16):
                    dstg_v[0, pl.ds(k * 16, 16)] = ho_v[pl.ds(j * CH + k * 16, 16)]
                pltpu.async_copy(rows_v.at[0], out_hbm.at[dstg_v.at[0]],
                                 ss[0]).wait()
                return carry

            nhc = (cnt + CH - 1) // CH
            lax.fori_loop(0, nhc, hbody, jnp.int32(0))

    out = edge_agg(xw_flat, gidx_p, dst_p, mp_pad, zeros_blk, history_buffer)
    return (out, out)


# speedup vs baseline: 21.4034x; 1.0591x over previous
import functools

import jax
import jax.numpy as jnp
from jax import lax
from jax.experimental import pallas as pl
from jax.experimental.pallas import tpu as pltpu
from jax.experimental.pallas import tpu_sc as plsc


def _mm_body(rp, d_out, x_ref, w_ref, b_ref, o_ref):
    res = jnp.dot(x_ref[...], w_ref[...], preferred_element_type=jnp.float32)
    for k in range(rp):
        o_ref[k] = res[:, k * d_out:(k + 1) * d_out] + b_ref[k, 0][None, :]


def kernel(x, edge_index, etypes, history_map, history_buffer, history_size, W, loop_w, b):
    N, D_IN = x.shape
    R = W.shape[0]
    E = etypes.shape[0]
    H, D_OUT = history_buffer.shape
    RP = R + 1

    NC, NS = 2, 16
    CH = 128
    NBUF = 2
    TM = 2000
    NLOC = N // NC
    NACC = NLOC + 8
    TRASH = NLOC
    MT = 5120

    src = edge_index[0]
    dst = edge_index[1]

    Wflat = jnp.transpose(
        jnp.concatenate([W, loop_w[None]], axis=0), (1, 0, 2)
    ).reshape(D_IN, RP * D_OUT)
    bias3 = jnp.zeros((RP, 1, D_OUT), jnp.float32).at[R, 0].set(b)
    xw = pl.pallas_call(
        functools.partial(_mm_body, RP, D_OUT),
        grid=(N // TM,),
        in_specs=[
            pl.BlockSpec((TM, D_IN), lambda i: (i, 0)),
            pl.BlockSpec((D_IN, RP * D_OUT), lambda i: (0, 0)),
            pl.BlockSpec((RP, 1, D_OUT), lambda i: (0, 0, 0)),
        ],
        out_specs=pl.BlockSpec((RP, TM, D_OUT), lambda i: (0, i, 0)),
        out_shape=jax.ShapeDtypeStruct((RP, N, D_OUT), jnp.float32),
    )(x, Wflat, bias3)
    xw_flat = xw.reshape((RP * N, D_OUT))

    nch = -(-(-(-E // (NS * CH))) // NBUF) * NBUF
    ncw = nch * CH
    Epad = ncw * NS
    pad = Epad - E
    gidx = etypes * N + src
    gidx_p = jnp.concatenate([gidx, jnp.zeros((pad,), jnp.int32)]).reshape(NS, ncw)
    dst_p = jnp.concatenate([dst, jnp.full((pad,), -1, jnp.int32)]).reshape(NS, ncw)
    hm_eff = jnp.where(history_size != 0, history_map, -1)
    mp_pad = jnp.concatenate(
        [hm_eff, jnp.full((NLOC + MT - N,), -1, jnp.int32)])
    ZCH = 320
    ZST = 312
    zeros_blk = jnp.zeros((ZCH, D_OUT), jnp.float32)
    G16 = ncw // 16
    HB = 448

    mesh = plsc.VectorSubcoreMesh(
        core_axis_name="c", subcore_axis_name="s", num_cores=NC, num_subcores=NS
    )

    @functools.partial(
        pl.kernel,
        out_type=jax.ShapeDtypeStruct((N, D_OUT), jnp.float32),
        mesh=mesh,
        compiler_params=pltpu.CompilerParams(needs_layout_passes=False),
        scratch_types=[
            pltpu.VMEM((ncw + CH,), jnp.int32),
            pltpu.VMEM((ncw + CH,), jnp.int32),
            pltpu.VMEM((MT,), jnp.int32),
            pltpu.VMEM((NBUF, CH), jnp.int32),
            pltpu.VMEM((NBUF, CH, D_OUT), jnp.float32),
            pltpu.VMEM((HB,), jnp.int32),
            pltpu.VMEM((HB,), jnp.int32),
            pltpu.VMEM_SHARED((NACC, D_OUT), jnp.float32),
            pltpu.SemaphoreType.DMA,
            pltpu.SemaphoreType.DMA,
            pltpu.SemaphoreType.DMA,
        ]
        + [pltpu.SemaphoreType.DMA] * (2 * NBUF),
    )
    def edge_agg(xw_hbm, gidx_hbm, dst_hbm, mp_hbm, z_hbm, hist_hbm, out_hbm,
                 gi_v, di_v, map_v, dstg_v, rows_v, hi_v, ho_v, acc_sh,
                 si0, si1, si2, *sems):
        sg = sems[:NBUF]
        ss = sems[NBUF:]
        c = lax.axis_index("c")
        s = lax.axis_index("s")
        lo = c * NLOC
        cp0 = pltpu.async_copy(gidx_hbm.at[s], gi_v.at[pl.ds(0, ncw)], si0)
        cp1 = pltpu.async_copy(dst_hbm.at[s], di_v.at[pl.ds(0, ncw)], si1)
        cp2 = pltpu.async_copy(mp_hbm.at[pl.ds(lo, MT)], map_v, si2)
        zbase = ZST * s
        pltpu.sync_copy(z_hbm, acc_sh.at[pl.ds(zbase, ZCH)])
        cp0.wait()
        cp1.wait()
        cp2.wait()

        def comp(g, off):
            d = di_v[pl.ds(g * 16, 16)]
            gx = gi_v[pl.ds(g * 16, 16)]
            dl = d - lo
            inr = (dl >= 0) & (dl < NLOC)
            dls = jnp.where(inr, dl, 0)
            hv = plsc.load_gather(map_v, [dls])
            keep = inr & (hv == -1)
            plsc.store_compressed(di_v.at[pl.ds(off, 16)], dl, mask=keep)
            plsc.store_compressed(gi_v.at[pl.ds(off, 16)], gx, mask=keep)
            return off + jnp.sum(keep.astype(jnp.int32))

        off = lax.fori_loop(0, G16, comp, jnp.int32(0))
        for k in range(CH // 16):
            gi_v[pl.ds(off + k * 16, 16)] = jnp.zeros((16,), jnp.int32)
            di_v[pl.ds(off + k * 16, 16)] = jnp.full((16,), TRASH, jnp.int32)
        ncc = (off + CH - 1) // CH
        plsc.subcore_barrier()

        def fire_gather(j, bslot):
            pltpu.async_copy(xw_hbm.at[gi_v.at[pl.ds(j * CH, CH)]],
                             rows_v.at[bslot], sg[bslot])

        def fire_scatter(j, bslot):
            for k in range(CH // 16):
                dstg_v[bslot, pl.ds(k * 16, 16)] = di_v[pl.ds(j * CH + k * 16, 16)]
            pltpu.async_copy(rows_v.at[bslot], acc_sh.at[dstg_v.at[bslot]],
                             ss[bslot], add=True)

        def wait_g(bslot):
            pltpu.make_async_copy(xw_hbm.at[pl.ds(0, CH)], rows_v.at[bslot],
                                  sg[bslot]).wait()

        def wait_s(bslot):
            pltpu.make_async_copy(xw_hbm.at[pl.ds(0, CH)], rows_v.at[bslot],
                                  ss[bslot]).wait()

        for bslot in range(NBUF):
            @pl.when(bslot < ncc)
            def _(bslot=bslot):
                fire_gather(bslot, bslot)

        def body(blk, carry):
            for bslot in range(NBUF):
                j = (blk - 1) * NBUF + bslot

                @pl.when(j < ncc)
                def _(j=j, bslot=bslot):
                    wait_g(bslot)
                    fire_scatter(j, bslot)
            for bslot in range(NBUF):
                j = blk * NBUF + bslot

                @pl.when(j < ncc)
                def _(j=j, bslot=bslot):
                    wait_s(bslot)
                    fire_gather(j, bslot)
            return carry

        nb = (ncc + NBUF - 1) // NBUF
        lax.fori_loop(1, nb, body, jnp.int32(0))
        base = (nb - 1) * NBUF
        for bslot in range(NBUF):
            j = base + bslot

            @pl.when((j >= 0) & (j < ncc))
            def _(j=j, bslot=bslot):
                wait_g(bslot)
                fire_scatter(j, bslot)
        for bslot in range(NBUF):
            j = base + bslot

            @pl.when((j >= 0) & (j < ncc))
            def _(j=j, bslot=bslot):
                wait_s(bslot)
        plsc.subcore_barrier()

        g0 = lo + zbase
        for q in range(ZCH // 80):
            pltpu.sync_copy(acc_sh.at[pl.ds(zbase + 80 * q, 80)],
                            rows_v.at[0, pl.ds(0, 80)])
            pltpu.sync_copy(xw_hbm.at[pl.ds(R * N + g0 + 80 * q, 80)],
                            rows_v.at[1, pl.ds(0, 80)])

            def addrow(i, carry):
                for l in range(D_OUT // 16):
                    rows_v[0, i, pl.ds(l * 16, 16)] = (
                        rows_v[0, i, pl.ds(l * 16, 16)]
                        + rows_v[1, i, pl.ds(l * 16, 16)]
                    )
                return carry

            lax.fori_loop(0, 80, addrow, jnp.int32(0))
            pltpu.sync_copy(rows_v.at[0, pl.ds(0, 80)],
                            out_hbm.at[pl.ds(g0 + 80 * q, 80)])
        plsc.subcore_barrier()

        def hcomp(g, cnt):
            m = map_v[pl.ds(zbase + g * 16, 16)]
            rowv = g0 + g * 16 + lax.iota(jnp.int32, 16)
            vm = m != -1
            plsc.store_compressed(hi_v.at[pl.ds(cnt, 16)], m, mask=vm)
            plsc.store_compressed(ho_v.at[pl.ds(cnt, 16)], rowv, mask=vm)
            return cnt + jnp.sum(vm.astype(jnp.int32))

        cnt = lax.fori_loop(0, ZCH // 16, hcomp, jnp.int32(0))

        @pl.when(cnt > 0)
        def _():
            lasth = hi_v[pl.ds(cnt - 1, 16)][0]
            lasto = ho_v[pl.ds(cnt - 1, 16)][0]
            for k in range(CH // 16):
                hi_v[pl.ds(cnt + k * 16, 16)] = jnp.full((16,), 0, jnp.int32) + lasth
                ho_v[pl.ds(cnt + k * 16, 16)] = jnp.full((16,), 0, jnp.int32) + lasto

            def hbody(j, carry):
                cph = pltpu.async_copy(
                    hist_hbm.at[hi_v.at[pl.ds(j * CH, CH)]], rows_v.at[0], sg[0])
                cph.wait()
                for k in range(CH // 16):
                    dstg_v[0, pl.ds(k * 16, 16)] = ho_v[pl.ds(j * CH + k * 16, 16)]
                pltpu.async_copy(rows_v.at[0], out_hbm.at[dstg_v.at[0]],
                                 ss[0]).wait()
                return carry

            nhc = (cnt + CH - 1) // CH
            lax.fori_loop(0, nhc, hbody, jnp.int32(0))

    out = edge_agg(xw_flat, gidx_p, dst_p, mp_pad, zeros_blk, history_buffer)
    return (out, out)
